# Initial kernel scaffold; baseline (speedup 1.0000x reference)
import jax
import jax.numpy as jnp
from jax.experimental import pallas as pl


def kernel(x, edge_index, W1, b1, W2, b2, W3, b3, Wp, bp):
    raise NotImplementedError("write your pallas kernel here")


# speedup vs baseline: 10.9100x; 10.9100x over previous
import functools

import jax
import jax.numpy as jnp
from jax import lax
from jax.experimental import pallas as pl
from jax.experimental.pallas import tpu as pltpu
from jax.experimental.pallas import tpu_sc as plsc

N = 10000
E = 320000
D_IN = 128
H = 64

NC = 2
NS = 16
NW = NC * NS

NP = 10240
EP_TOT = 327680
EPT = EP_TOT // NW
C = 128
RPT = NP // NS
ZR = 64

_mesh = functools.partial(
    plsc.VectorSubcoreMesh, core_axis_name="c", subcore_axis_name="s"
)


def _make_deg_kernel():
    @functools.partial(
        pl.kernel,
        out_type=jax.ShapeDtypeStruct((NC, NP, 16), jnp.float32),
        mesh=_mesh(),
        compiler_params=pltpu.CompilerParams(use_tc_tiling_on_sc=False),
        scratch_types=[
            pltpu.VMEM((C,), jnp.int32),
            pltpu.VMEM((C, 16), jnp.float32),
            pltpu.VMEM((ZR, 16), jnp.float32),
            pltpu.VMEM_SHARED((NP, 16), jnp.float32),
        ],
    )
    def deg_kernel(dst_hbm, out_hbm, didx, ones, zbuf, accum):
        c = lax.axis_index("c")
        s = lax.axis_index("s")
        tid = c * NS + s
        base = tid * EPT

        zv = jnp.zeros((16,), jnp.float32)
        ov = jnp.ones((16,), jnp.float32)

        def fill(i, _):
            zbuf[i, :] = zv
            ones[i, :] = ov
            return 0

        lax.fori_loop(0, ZR, fill, 0)

        def fill2(i, _):
            ones[ZR + i, :] = ov
            return 0

        lax.fori_loop(0, C - ZR, fill2, 0)

        def zero_accum(j, _):
            pltpu.sync_copy(zbuf, accum.at[pl.ds(s * RPT + j * ZR, ZR)])
            return 0

        lax.fori_loop(0, RPT // ZR, zero_accum, 0)
        plsc.subcore_barrier()

        def body(i, _):
            pltpu.sync_copy(dst_hbm.at[pl.ds(base + i * C, C)], didx)
            pltpu.sync_copy(ones, accum.at[didx], add=True)
            return 0

        lax.fori_loop(0, EPT // C, body, 0)
        plsc.subcore_barrier()

        pltpu.sync_copy(
            accum.at[pl.ds(s * RPT, RPT)], out_hbm.at[c, pl.ds(s * RPT, RPT)]
        )

    return deg_kernel


def _make_edge_kernel(F):
    @functools.partial(
        pl.kernel,
        out_type=jax.ShapeDtypeStruct((NC, NP, F), jnp.float32),
        mesh=_mesh(),
        compiler_params=pltpu.CompilerParams(use_tc_tiling_on_sc=False),
        scratch_types=[
            pltpu.VMEM((C,), jnp.int32),
            pltpu.VMEM((C,), jnp.int32),
            pltpu.VMEM((C, F), jnp.float32),
            pltpu.VMEM((ZR, F), jnp.float32),
            pltpu.VMEM_SHARED((NP, F), jnp.float32),
            pltpu.SemaphoreType.DMA,
        ],
    )
    def edge_kernel(g_hbm, src_hbm, dst_hbm, out_hbm, sidx, didx, rows, zbuf,
                    accum, sem):
        c = lax.axis_index("c")
        s = lax.axis_index("s")
        tid = c * NS + s
        base = tid * EPT

        zv = jnp.zeros((16,), jnp.float32)

        def fill(i, _):
            for k in range(F // 16):
                zbuf[i, pl.ds(k * 16, 16)] = zv
            return 0

        lax.fori_loop(0, ZR, fill, 0)

        def zero_accum(j, _):
            pltpu.sync_copy(zbuf, accum.at[pl.ds(s * RPT + j * ZR, ZR)])
            return 0

        lax.fori_loop(0, RPT // ZR, zero_accum, 0)
        plsc.subcore_barrier()

        def body(i, _):
            e0 = base + i * C
            pltpu.sync_copy(src_hbm.at[pl.ds(e0, C)], sidx)
            pltpu.async_copy(g_hbm.at[sidx], rows, sem).wait()
            pltpu.sync_copy(dst_hbm.at[pl.ds(e0, C)], didx)
            pltpu.sync_copy(rows, accum.at[didx], add=True)
            return 0

        lax.fori_loop(0, EPT // C, body, 0)
        plsc.subcore_barrier()

        pltpu.sync_copy(
            accum.at[pl.ds(s * RPT, RPT)], out_hbm.at[c, pl.ds(s * RPT, RPT)]
        )

    return edge_kernel


_deg_call = _make_deg_kernel()
_edge64_call = _make_edge_kernel(H)
_edge32_call = _make_edge_kernel(32)

_TC_R = 2048


def _tc1_body(x_ref, w_ref, d_ref, g_ref, dis_ref):
    deg = d_ref[:, 0:1] + d_ref[:, 1:2] + 1.0
    dis = lax.rsqrt(deg)
    h = jnp.dot(x_ref[...], w_ref[...], preferred_element_type=jnp.float32)
    g_ref[...] = h * dis
    dis_ref[...] = dis


def _tc1_call(xp, w1, dcols):
    grid = (NP // _TC_R,)
    return pl.pallas_call(
        _tc1_body,
        grid=grid,
        in_specs=[
            pl.BlockSpec((_TC_R, D_IN), lambda i: (i, 0)),
            pl.BlockSpec((D_IN, H), lambda i: (0, 0)),
            pl.BlockSpec((_TC_R, NC), lambda i: (i, 0)),
        ],
        out_specs=[
            pl.BlockSpec((_TC_R, H), lambda i: (i, 0)),
            pl.BlockSpec((_TC_R, 1), lambda i: (i, 0)),
        ],
        out_shape=[
            jax.ShapeDtypeStruct((NP, H), jnp.float32),
            jax.ShapeDtypeStruct((NP, 1), jnp.float32),
        ],
    )(xp, w1, dcols)


def _tcmid_body(p_ref, g_ref, dis_ref, b_ref, w_ref, o_ref):
    dis = dis_ref[...]
    u = (p_ref[0] + p_ref[1] + g_ref[...]) * dis + b_ref[...]
    t = jnp.maximum(u, 0.0)
    h = jnp.dot(t, w_ref[...], preferred_element_type=jnp.float32)
    o_ref[...] = h * dis


def _tcmid_call(p, g, dis, b, w):
    F = g.shape[1]
    F2 = w.shape[1]
    grid = (NP // _TC_R,)
    return pl.pallas_call(
        _tcmid_body,
        grid=grid,
        in_specs=[
            pl.BlockSpec((NC, _TC_R, F), lambda i: (0, i, 0)),
            pl.BlockSpec((_TC_R, F), lambda i: (i, 0)),
            pl.BlockSpec((_TC_R, 1), lambda i: (i, 0)),
            pl.BlockSpec((1, F), lambda i: (0, 0)),
            pl.BlockSpec((F, F2), lambda i: (0, 0)),
        ],
        out_specs=pl.BlockSpec((_TC_R, F2), lambda i: (i, 0)),
        out_shape=jax.ShapeDtypeStruct((NP, F2), jnp.float32),
    )(p, g, dis, b, w)


def _tcfin_body(p_ref, g_ref, dis_ref, b_ref, wp_ref, bp_ref, o_ref):
    u = (p_ref[0] + p_ref[1] + g_ref[...]) * dis_ref[...] + b_ref[...]
    t = jnp.maximum(u, 0.0)
    o_ref[...] = (
        jnp.dot(t, wp_ref[...], preferred_element_type=jnp.float32)
        + bp_ref[...]
    )


def _tcfin_call(p, g, dis, b, wp, bp):
    F = g.shape[1]
    grid = (NP // _TC_R,)
    return pl.pallas_call(
        _tcfin_body,
        grid=grid,
        in_specs=[
            pl.BlockSpec((NC, _TC_R, F), lambda i: (0, i, 0)),
            pl.BlockSpec((_TC_R, F), lambda i: (i, 0)),
            pl.BlockSpec((_TC_R, 1), lambda i: (i, 0)),
            pl.BlockSpec((1, F), lambda i: (0, 0)),
            pl.BlockSpec((F, 1), lambda i: (0, 0)),
            pl.BlockSpec((1, 1), lambda i: (0, 0)),
        ],
        out_specs=pl.BlockSpec((_TC_R, 1), lambda i: (i, 0)),
        out_shape=jax.ShapeDtypeStruct((NP, 1), jnp.float32),
    )(p, g, dis, b, wp, bp)


def kernel(x, edge_index, W1, b1, W2, b2, W3, b3, Wp, bp):
    src = edge_index[0]
    dst = edge_index[1]
    pad_e = EP_TOT - E
    fill = jnp.full((pad_e,), N, jnp.int32)
    srcp = jnp.concatenate([src, fill])
    dstp = jnp.concatenate([dst, fill])
    xp = jnp.pad(x, ((0, NP - N), (0, 0)))

    degp = _deg_call(dstp)
    dcols = jnp.transpose(degp[:, :, 0])

    g1, dis = _tc1_call(xp, W1, dcols)
    p1 = _edge64_call(g1, srcp, dstp)
    g2 = _tcmid_call(p1, g1, dis, b1.reshape(1, H), W2)
    p2 = _edge64_call(g2, srcp, dstp)
    g3 = _tcmid_call(p2, g2, dis, b2.reshape(1, H), W3)
    p3 = _edge32_call(g3, srcp, dstp)
    out = _tcfin_call(p3, g3, dis, b3.reshape(1, 32), Wp, bp.reshape(1, 1))
    return out[:N]


# speedup vs baseline: 14.8110x; 1.3576x over previous
import functools

import jax
import jax.numpy as jnp
from jax import lax
from jax.experimental import pallas as pl
from jax.experimental.pallas import tpu as pltpu
from jax.experimental.pallas import tpu_sc as plsc

N = 10000
E = 320000
D_IN = 128
H = 64

NC = 2
NS = 16
NW = NC * NS

NP = 10240
EP_TOT = 327680
EPT = EP_TOT // NW
C = 128
NCH = EPT // C
NPAIR = NCH // 2
RPT = NP // NS
ZR = 64

_mesh = functools.partial(
    plsc.VectorSubcoreMesh, core_axis_name="c", subcore_axis_name="s"
)


def _make_deg_kernel():
    @functools.partial(
        pl.kernel,
        out_type=jax.ShapeDtypeStruct((NC, NP, 16), jnp.float32),
        mesh=_mesh(),
        compiler_params=pltpu.CompilerParams(use_tc_tiling_on_sc=False),
        scratch_types=[
            pltpu.VMEM((NCH, C), jnp.int32),
            pltpu.VMEM((C, 16), jnp.float32),
            pltpu.VMEM((ZR, 16), jnp.float32),
            pltpu.VMEM_SHARED((NP, 16), jnp.float32),
            pltpu.SemaphoreType.DMA,
            pltpu.SemaphoreType.DMA,
        ],
    )
    def deg_kernel(dst_hbm, out_hbm, didx, ones, zbuf, accum, sem0, sem1):
        c = lax.axis_index("c")
        s = lax.axis_index("s")
        tid = c * NS + s

        pltpu.sync_copy(dst_hbm.at[pl.ds(tid * NCH, NCH)], didx)

        zv = jnp.zeros((16,), jnp.float32)
        ov = jnp.ones((16,), jnp.float32)

        def fill(i, _):
            zbuf[i, :] = zv
            ones[i, :] = ov
            ones[ZR + i, :] = ov
            return 0

        lax.fori_loop(0, ZR, fill, 0)

        def zero_accum(j, _):
            pltpu.sync_copy(zbuf, accum.at[pl.ds(s * RPT + j * ZR, ZR)])
            return 0

        lax.fori_loop(0, RPT // ZR, zero_accum, 0)
        plsc.subcore_barrier()

        pltpu.async_copy(ones, accum.at[didx.at[0]], sem0, add=True)

        def body(jj, _):
            j0 = 2 * jj
            pltpu.async_copy(ones, accum.at[didx.at[j0 + 1]], sem1, add=True)
            pltpu.make_async_copy(ones, accum.at[didx.at[j0]], sem0).wait()

            @pl.when(jj < NPAIR - 1)
            def _():
                pltpu.async_copy(ones, accum.at[didx.at[j0 + 2]], sem0,
                                 add=True)

            pltpu.make_async_copy(ones, accum.at[didx.at[j0 + 1]], sem1).wait()
            return 0

        lax.fori_loop(0, NPAIR, body, 0)
        plsc.subcore_barrier()

        pltpu.sync_copy(
            accum.at[pl.ds(s * RPT, RPT)], out_hbm.at[c, pl.ds(s * RPT, RPT)]
        )

    return deg_kernel


def _make_edge_kernel(F):
    @functools.partial(
        pl.kernel,
        out_type=jax.ShapeDtypeStruct((NC, NP, F), jnp.float32),
        mesh=_mesh(),
        compiler_params=pltpu.CompilerParams(use_tc_tiling_on_sc=False),
        scratch_types=[
            pltpu.VMEM((NCH, C), jnp.int32),
            pltpu.VMEM((NCH, C), jnp.int32),
            pltpu.VMEM((2, C, F), jnp.float32),
            pltpu.VMEM((ZR, F), jnp.float32),
            pltpu.VMEM_SHARED((NP, F), jnp.float32),
            pltpu.SemaphoreType.DMA,
            pltpu.SemaphoreType.DMA,
        ],
    )
    def edge_kernel(g_hbm, src_hbm, dst_hbm, out_hbm, sidx, didx, rows, zbuf,
                    accum, sem0, sem1):
        c = lax.axis_index("c")
        s = lax.axis_index("s")
        tid = c * NS + s

        pltpu.sync_copy(src_hbm.at[pl.ds(tid * NCH, NCH)], sidx)
        pltpu.sync_copy(dst_hbm.at[pl.ds(tid * NCH, NCH)], didx)

        zv = jnp.zeros((16,), jnp.float32)

        def fill(i, _):
            for k in range(F // 16):
                zbuf[i, pl.ds(k * 16, 16)] = zv
            return 0

        lax.fori_loop(0, ZR, fill, 0)

        def zero_accum(j, _):
            pltpu.sync_copy(zbuf, accum.at[pl.ds(s * RPT + j * ZR, ZR)])
            return 0

        lax.fori_loop(0, RPT // ZR, zero_accum, 0)
        plsc.subcore_barrier()

        buf0 = rows.at[0]
        buf1 = rows.at[1]
        pltpu.async_copy(g_hbm.at[sidx.at[0]], buf0, sem0)

        def body(jj, _):
            j0 = 2 * jj
            pltpu.make_async_copy(g_hbm.at[sidx.at[j0]], buf0, sem0).wait()
            pltpu.async_copy(g_hbm.at[sidx.at[j0 + 1]], buf1, sem1)
            pltpu.sync_copy(buf0, accum.at[didx.at[j0]], add=True)
            pltpu.make_async_copy(g_hbm.at[sidx.at[j0 + 1]], buf1, sem1).wait()

            @pl.when(jj < NPAIR - 1)
            def _():
                pltpu.async_copy(g_hbm.at[sidx.at[j0 + 2]], buf0, sem0)

            pltpu.sync_copy(buf1, accum.at[didx.at[j0 + 1]], add=True)
            return 0

        lax.fori_loop(0, NPAIR, body, 0)
        plsc.subcore_barrier()

        pltpu.sync_copy(
            accum.at[pl.ds(s * RPT, RPT)], out_hbm.at[c, pl.ds(s * RPT, RPT)]
        )

    return edge_kernel


_deg_call = _make_deg_kernel()
_edge64_call = _make_edge_kernel(H)
_edge32_call = _make_edge_kernel(32)

_TC_R = 2048


def _tc1_body(x_ref, w_ref, d_ref, g_ref, dis_ref):
    deg = d_ref[:, 0:1] + d_ref[:, 1:2] + 1.0
    dis = lax.rsqrt(deg)
    h = jnp.dot(x_ref[...], w_ref[...], preferred_element_type=jnp.float32)
    g_ref[...] = h * dis
    dis_ref[...] = dis


def _tc1_call(xp, w1, dcols):
    grid = (NP // _TC_R,)
    return pl.pallas_call(
        _tc1_body,
        grid=grid,
        in_specs=[
            pl.BlockSpec((_TC_R, D_IN), lambda i: (i, 0)),
            pl.BlockSpec((D_IN, H), lambda i: (0, 0)),
            pl.BlockSpec((_TC_R, NC), lambda i: (i, 0)),
        ],
        out_specs=[
            pl.BlockSpec((_TC_R, H), lambda i: (i, 0)),
            pl.BlockSpec((_TC_R, 1), lambda i: (i, 0)),
        ],
        out_shape=[
            jax.ShapeDtypeStruct((NP, H), jnp.float32),
            jax.ShapeDtypeStruct((NP, 1), jnp.float32),
        ],
    )(xp, w1, dcols)


def _tcmid_body(p_ref, g_ref, dis_ref, b_ref, w_ref, o_ref):
    dis = dis_ref[...]
    u = (p_ref[0] + p_ref[1] + g_ref[...]) * dis + b_ref[...]
    t = jnp.maximum(u, 0.0)
    h = jnp.dot(t, w_ref[...], preferred_element_type=jnp.float32)
    o_ref[...] = h * dis


def _tcmid_call(p, g, dis, b, w):
    F = g.shape[1]
    F2 = w.shape[1]
    grid = (NP // _TC_R,)
    return pl.pallas_call(
        _tcmid_body,
        grid=grid,
        in_specs=[
            pl.BlockSpec((NC, _TC_R, F), lambda i: (0, i, 0)),
            pl.BlockSpec((_TC_R, F), lambda i: (i, 0)),
            pl.BlockSpec((_TC_R, 1), lambda i: (i, 0)),
            pl.BlockSpec((1, F), lambda i: (0, 0)),
            pl.BlockSpec((F, F2), lambda i: (0, 0)),
        ],
        out_specs=pl.BlockSpec((_TC_R, F2), lambda i: (i, 0)),
        out_shape=jax.ShapeDtypeStruct((NP, F2), jnp.float32),
    )(p, g, dis, b, w)


def _tcfin_body(p_ref, g_ref, dis_ref, b_ref, wp_ref, bp_ref, o_ref):
    u = (p_ref[0] + p_ref[1] + g_ref[...]) * dis_ref[...] + b_ref[...]
    t = jnp.maximum(u, 0.0)
    o_ref[...] = (
        jnp.dot(t, wp_ref[...], preferred_element_type=jnp.float32)
        + bp_ref[...]
    )


def _tcfin_call(p, g, dis, b, wp, bp):
    F = g.shape[1]
    grid = (NP // _TC_R,)
    return pl.pallas_call(
        _tcfin_body,
        grid=grid,
        in_specs=[
            pl.BlockSpec((NC, _TC_R, F), lambda i: (0, i, 0)),
            pl.BlockSpec((_TC_R, F), lambda i: (i, 0)),
            pl.BlockSpec((_TC_R, 1), lambda i: (i, 0)),
            pl.BlockSpec((1, F), lambda i: (0, 0)),
            pl.BlockSpec((F, 1), lambda i: (0, 0)),
            pl.BlockSpec((1, 1), lambda i: (0, 0)),
        ],
        out_specs=pl.BlockSpec((_TC_R, 1), lambda i: (i, 0)),
        out_shape=jax.ShapeDtypeStruct((NP, 1), jnp.float32),
    )(p, g, dis, b, wp, bp)


def kernel(x, edge_index, W1, b1, W2, b2, W3, b3, Wp, bp):
    src = edge_index[0]
    dst = edge_index[1]
    pad_e = EP_TOT - E
    fill = jnp.full((pad_e,), N, jnp.int32)
    srcp = jnp.concatenate([src, fill]).reshape(EP_TOT // C, C)
    dstp = jnp.concatenate([dst, fill]).reshape(EP_TOT // C, C)
    xp = jnp.pad(x, ((0, NP - N), (0, 0)))

    degp = _deg_call(dstp)
    dcols = jnp.transpose(degp[:, :, 0])

    g1, dis = _tc1_call(xp, W1, dcols)
    p1 = _edge64_call(g1, srcp, dstp)
    g2 = _tcmid_call(p1, g1, dis, b1.reshape(1, H), W2)
    p2 = _edge64_call(g2, srcp, dstp)
    g3 = _tcmid_call(p2, g2, dis, b2.reshape(1, H), W3)
    p3 = _edge32_call(g3, srcp, dstp)
    out = _tcfin_call(p3, g3, dis, b3.reshape(1, 32), Wp, bp.reshape(1, 1))
    return out[:N]


# speedup vs baseline: 16.6901x; 1.1269x over previous
import functools

import jax
import jax.numpy as jnp
from jax import lax
from jax.experimental import pallas as pl
from jax.experimental.pallas import tpu as pltpu
from jax.experimental.pallas import tpu_sc as plsc

N = 10000
E = 320000
D_IN = 128
H = 64

NC = 2
NS = 16
NW = NC * NS

NP = 10240
EP_TOT = 327680
EPT = EP_TOT // NW
C = 128
NCH = EPT // C
NPAIR = NCH // 2
NB = 8
NGRP = NCH // NB
RPT = NP // NS
ZR = 64

_mesh = functools.partial(
    plsc.VectorSubcoreMesh, core_axis_name="c", subcore_axis_name="s"
)


def _make_deg_kernel():
    @functools.partial(
        pl.kernel,
        out_type=jax.ShapeDtypeStruct((NC, NP, 16), jnp.float32),
        mesh=_mesh(),
        compiler_params=pltpu.CompilerParams(use_tc_tiling_on_sc=False),
        scratch_types=[
            pltpu.VMEM((NCH, C), jnp.int32),
            pltpu.VMEM((C, 16), jnp.float32),
            pltpu.VMEM((ZR, 16), jnp.float32),
            pltpu.VMEM_SHARED((NP, 16), jnp.float32),
            pltpu.SemaphoreType.DMA,
            pltpu.SemaphoreType.DMA,
        ],
    )
    def deg_kernel(dst_hbm, out_hbm, didx, ones, zbuf, accum, sem0, sem1):
        c = lax.axis_index("c")
        s = lax.axis_index("s")
        tid = c * NS + s

        pltpu.sync_copy(dst_hbm.at[pl.ds(tid * NCH, NCH)], didx)

        zv = jnp.zeros((16,), jnp.float32)
        ov = jnp.ones((16,), jnp.float32)

        def fill(i, _):
            zbuf[i, :] = zv
            ones[i, :] = ov
            ones[ZR + i, :] = ov
            return 0

        lax.fori_loop(0, ZR, fill, 0)

        def zero_accum(j, _):
            pltpu.sync_copy(zbuf, accum.at[pl.ds(s * RPT + j * ZR, ZR)])
            return 0

        lax.fori_loop(0, RPT // ZR, zero_accum, 0)
        plsc.subcore_barrier()

        pltpu.async_copy(ones, accum.at[didx.at[0]], sem0, add=True)

        def body(jj, _):
            j0 = 2 * jj
            pltpu.async_copy(ones, accum.at[didx.at[j0 + 1]], sem1, add=True)
            pltpu.make_async_copy(ones, accum.at[didx.at[j0]], sem0).wait()

            @pl.when(jj < NPAIR - 1)
            def _():
                pltpu.async_copy(ones, accum.at[didx.at[j0 + 2]], sem0,
                                 add=True)

            pltpu.make_async_copy(ones, accum.at[didx.at[j0 + 1]], sem1).wait()
            return 0

        lax.fori_loop(0, NPAIR, body, 0)
        plsc.subcore_barrier()

        pltpu.sync_copy(
            accum.at[pl.ds(s * RPT, RPT)], out_hbm.at[c, pl.ds(s * RPT, RPT)]
        )

    return deg_kernel


def _make_edge_kernel(F):
    @functools.partial(
        pl.kernel,
        out_type=jax.ShapeDtypeStruct((NC, NP, F), jnp.float32),
        mesh=_mesh(),
        compiler_params=pltpu.CompilerParams(use_tc_tiling_on_sc=False),
        scratch_types=[
            pltpu.VMEM((NCH, C), jnp.int32),
            pltpu.VMEM((NCH, C), jnp.int32),
            pltpu.VMEM((NB, C, F), jnp.float32),
            pltpu.VMEM((ZR, F), jnp.float32),
            pltpu.VMEM_SHARED((NP, F), jnp.float32),
            [pltpu.SemaphoreType.DMA] * NB,
            [pltpu.SemaphoreType.DMA] * NB,
        ],
    )
    def edge_kernel(g_hbm, src_hbm, dst_hbm, out_hbm, sidx, didx, rows, zbuf,
                    accum, gsems, ssems):
        c = lax.axis_index("c")
        s = lax.axis_index("s")
        tid = c * NS + s

        pltpu.sync_copy(src_hbm.at[pl.ds(tid * NCH, NCH)], sidx)
        pltpu.sync_copy(dst_hbm.at[pl.ds(tid * NCH, NCH)], didx)

        zv = jnp.zeros((16,), jnp.float32)

        def fill(i, _):
            for k in range(F // 16):
                zbuf[i, pl.ds(k * 16, 16)] = zv
            return 0

        lax.fori_loop(0, ZR, fill, 0)

        def zero_accum(j, _):
            pltpu.sync_copy(zbuf, accum.at[pl.ds(s * RPT + j * ZR, ZR)])
            return 0

        lax.fori_loop(0, RPT // ZR, zero_accum, 0)
        plsc.subcore_barrier()

        for b in range(NB - 2):
            pltpu.async_copy(g_hbm.at[sidx.at[b]], rows.at[b], gsems[b])

        def body(gg, _):
            j0 = gg * NB
            for b in range(NB):
                bw = (b - 2) % NB

                def wait_scat():
                    pltpu.make_async_copy(
                        rows.at[bw], accum.at[didx.at[0]], ssems[bw]
                    ).wait()

                if b >= 2:
                    wait_scat()
                    jg = j0 + NB + b - 2

                    @pl.when(gg < NGRP - 1)
                    def _():
                        pltpu.async_copy(
                            g_hbm.at[sidx.at[jg]], rows.at[bw], gsems[bw]
                        )
                else:
                    @pl.when(gg > 0)
                    def _():
                        wait_scat()

                    jg = j0 + NB - 2 + b
                    pltpu.async_copy(
                        g_hbm.at[sidx.at[jg]], rows.at[bw], gsems[bw]
                    )

                j = j0 + b
                pltpu.make_async_copy(
                    g_hbm.at[sidx.at[j]], rows.at[b], gsems[b]
                ).wait()
                pltpu.async_copy(
                    rows.at[b], accum.at[didx.at[j]], ssems[b], add=True
                )
            return 0

        lax.fori_loop(0, NGRP, body, 0)
        for b in (NB - 2, NB - 1):
            pltpu.make_async_copy(
                rows.at[b], accum.at[didx.at[0]], ssems[b]
            ).wait()
        plsc.subcore_barrier()

        pltpu.sync_copy(
            accum.at[pl.ds(s * RPT, RPT)], out_hbm.at[c, pl.ds(s * RPT, RPT)]
        )

    return edge_kernel


_deg_call = _make_deg_kernel()
_edge64_call = _make_edge_kernel(H)
_edge32_call = _make_edge_kernel(32)

_TC_R = 2048


def _tc1_body(x_ref, w_ref, d_ref, g_ref, dis_ref):
    deg = d_ref[:, 0:1] + d_ref[:, 1:2] + 1.0
    dis = lax.rsqrt(deg)
    h = jnp.dot(x_ref[...], w_ref[...], preferred_element_type=jnp.float32)
    g_ref[...] = h * dis
    dis_ref[...] = dis


def _tc1_call(xp, w1, dcols):
    grid = (NP // _TC_R,)
    return pl.pallas_call(
        _tc1_body,
        grid=grid,
        in_specs=[
            pl.BlockSpec((_TC_R, D_IN), lambda i: (i, 0)),
            pl.BlockSpec((D_IN, H), lambda i: (0, 0)),
            pl.BlockSpec((_TC_R, NC), lambda i: (i, 0)),
        ],
        out_specs=[
            pl.BlockSpec((_TC_R, H), lambda i: (i, 0)),
            pl.BlockSpec((_TC_R, 1), lambda i: (i, 0)),
        ],
        out_shape=[
            jax.ShapeDtypeStruct((NP, H), jnp.float32),
            jax.ShapeDtypeStruct((NP, 1), jnp.float32),
        ],
    )(xp, w1, dcols)


def _tcmid_body(p_ref, g_ref, dis_ref, b_ref, w_ref, o_ref):
    dis = dis_ref[...]
    u = (p_ref[0] + p_ref[1] + g_ref[...]) * dis + b_ref[...]
    t = jnp.maximum(u, 0.0)
    h = jnp.dot(t, w_ref[...], preferred_element_type=jnp.float32)
    o_ref[...] = h * dis


def _tcmid_call(p, g, dis, b, w):
    F = g.shape[1]
    F2 = w.shape[1]
    grid = (NP // _TC_R,)
    return pl.pallas_call(
        _tcmid_body,
        grid=grid,
        in_specs=[
            pl.BlockSpec((NC, _TC_R, F), lambda i: (0, i, 0)),
            pl.BlockSpec((_TC_R, F), lambda i: (i, 0)),
            pl.BlockSpec((_TC_R, 1), lambda i: (i, 0)),
            pl.BlockSpec((1, F), lambda i: (0, 0)),
            pl.BlockSpec((F, F2), lambda i: (0, 0)),
        ],
        out_specs=pl.BlockSpec((_TC_R, F2), lambda i: (i, 0)),
        out_shape=jax.ShapeDtypeStruct((NP, F2), jnp.float32),
    )(p, g, dis, b, w)


def _tcfin_body(p_ref, g_ref, dis_ref, b_ref, wp_ref, bp_ref, o_ref):
    u = (p_ref[0] + p_ref[1] + g_ref[...]) * dis_ref[...] + b_ref[...]
    t = jnp.maximum(u, 0.0)
    o_ref[...] = (
        jnp.dot(t, wp_ref[...], preferred_element_type=jnp.float32)
        + bp_ref[...]
    )


def _tcfin_call(p, g, dis, b, wp, bp):
    F = g.shape[1]
    grid = (NP // _TC_R,)
    return pl.pallas_call(
        _tcfin_body,
        grid=grid,
        in_specs=[
            pl.BlockSpec((NC, _TC_R, F), lambda i: (0, i, 0)),
            pl.BlockSpec((_TC_R, F), lambda i: (i, 0)),
            pl.BlockSpec((_TC_R, 1), lambda i: (i, 0)),
            pl.BlockSpec((1, F), lambda i: (0, 0)),
            pl.BlockSpec((F, 1), lambda i: (0, 0)),
            pl.BlockSpec((1, 1), lambda i: (0, 0)),
        ],
        out_specs=pl.BlockSpec((_TC_R, 1), lambda i: (i, 0)),
        out_shape=jax.ShapeDtypeStruct((NP, 1), jnp.float32),
    )(p, g, dis, b, wp, bp)


def kernel(x, edge_index, W1, b1, W2, b2, W3, b3, Wp, bp):
    src = edge_index[0]
    dst = edge_index[1]
    pad_e = EP_TOT - E
    fill = jnp.full((pad_e,), N, jnp.int32)
    srcp = jnp.concatenate([src, fill]).reshape(EP_TOT // C, C)
    dstp = jnp.concatenate([dst, fill]).reshape(EP_TOT // C, C)
    xp = jnp.pad(x, ((0, NP - N), (0, 0)))

    degp = _deg_call(dstp)
    dcols = jnp.transpose(degp[:, :, 0])

    g1, dis = _tc1_call(xp, W1, dcols)
    p1 = _edge64_call(g1, srcp, dstp)
    g2 = _tcmid_call(p1, g1, dis, b1.reshape(1, H), W2)
    p2 = _edge64_call(g2, srcp, dstp)
    g3 = _tcmid_call(p2, g2, dis, b2.reshape(1, H), W3)
    p3 = _edge32_call(g3, srcp, dstp)
    out = _tcfin_call(p3, g3, dis, b3.reshape(1, 32), Wp, bp.reshape(1, 1))
    return out[:N]
